# baseline (speedup 1.0000x reference)
import jax
import jax.numpy as jnp
from jax import lax
from jax.experimental import pallas as pl
from jax.experimental.pallas import tpu as pltpu
from jax.experimental.pallas import tpu_sc as plsc

VOCAB = 100000
EMBED_DIM = 1024
BATCH = 4
SEQ_LEN = 4096

_INFO = plsc.get_sparse_core_info()
NC, NS = _INFO.num_cores, _INFO.num_subcores
NW = NC * NS
TOTAL = BATCH * SEQ_LEN
B_PER_W = TOTAL // NW
W_PER_B = SEQ_LEN // B_PER_W
CHUNK = 16
N_CHUNKS = B_PER_W // CHUNK
NBUF = 4
LOOKAHEAD = NBUF - 1


def _gather_body(table_hbm, idx_hbm, out_hbm, idx_v, rows_v, gsem, osem):
    wid = lax.axis_index("s") * NC + lax.axis_index("c")
    bat = wid // W_PER_B
    seq0 = (wid % W_PER_B) * B_PER_W

    pltpu.sync_copy(idx_hbm.at[bat, pl.ds(seq0, B_PER_W)], idx_v)

    def idx_slc(c):
        return idx_v.at[pl.ds(c * CHUNK, CHUNK)]

    def out_slc(c):
        return out_hbm.at[bat, pl.ds(seq0 + c * CHUNK, CHUNK)]

    def start_gather(c, b):
        pltpu.async_copy(table_hbm.at[idx_slc(c)], rows_v.at[b], gsem.at[b])

    def wait_gather(c, b):
        pltpu.make_async_copy(
            table_hbm.at[idx_slc(c)], rows_v.at[b], gsem.at[b]).wait()

    def start_wb(c, b):
        pltpu.async_copy(rows_v.at[b], out_slc(c), osem.at[b])

    def wait_wb(c, b):
        pltpu.make_async_copy(rows_v.at[b], out_slc(c), osem.at[b]).wait()

    for b in range(LOOKAHEAD):
        start_gather(b, b)

    start_gather(LOOKAHEAD, LOOKAHEAD % NBUF)
    wait_gather(0, 0)
    start_wb(0, 0)

    @pl.loop(1, N_CHUNKS - LOOKAHEAD, step=NBUF)
    def _steady(g):
        for b in range(NBUF):
            j = g + b
            bj = (1 + b) % NBUF
            bp = b
            wait_wb(j - 1, bp)
            start_gather(j + LOOKAHEAD, bp)
            wait_gather(j, bj)
            start_wb(j, bj)

    for j in range(N_CHUNKS - LOOKAHEAD, N_CHUNKS):
        wait_wb(j - 1, (j - 1) % NBUF)
        wait_gather(j, j % NBUF)
        start_wb(j, j % NBUF)
    wait_wb(N_CHUNKS - 1, (N_CHUNKS - 1) % NBUF)


@jax.jit
def kernel(input_ids, embedding_table):
    idx = input_ids.astype(jnp.int32)
    mesh = plsc.VectorSubcoreMesh(core_axis_name="c", subcore_axis_name="s")
    return pl.kernel(
        _gather_body,
        out_type=jax.ShapeDtypeStruct((BATCH, SEQ_LEN, EMBED_DIM), jnp.float32),
        mesh=mesh,
        scratch_types=[
            pltpu.VMEM((B_PER_W,), jnp.int32),
            pltpu.VMEM((NBUF, CHUNK, EMBED_DIM), jnp.float32),
            pltpu.SemaphoreType.DMA((NBUF,)),
            pltpu.SemaphoreType.DMA((NBUF,)),
        ],
    )(embedding_table, idx)


# speedup vs baseline: 1.0014x; 1.0014x over previous
import jax
import jax.numpy as jnp
from jax import lax
from jax.experimental import pallas as pl
from jax.experimental.pallas import tpu as pltpu
from jax.experimental.pallas import tpu_sc as plsc

VOCAB = 100000
EMBED_DIM = 1024
BATCH = 4
SEQ_LEN = 4096

_INFO = plsc.get_sparse_core_info()
NC, NS = _INFO.num_cores, _INFO.num_subcores
NW = NC * NS
TOTAL = BATCH * SEQ_LEN
B_PER_W = TOTAL // NW
W_PER_B = SEQ_LEN // B_PER_W
CHUNK = 16
N_CHUNKS = B_PER_W // CHUNK
NBUF = 4
LOOKAHEAD = NBUF - 1


def _gather_body(table_hbm, idx_hbm, out_hbm, idx_v, rows_v, gsem, osem, isem):
    wid = lax.axis_index("s") * NC + lax.axis_index("c")
    bat = wid // W_PER_B
    seq0 = (wid % W_PER_B) * B_PER_W

    HEAD = 128
    pltpu.sync_copy(idx_hbm.at[bat, pl.ds(seq0, HEAD)],
                    idx_v.at[pl.ds(0, HEAD)])
    pltpu.async_copy(idx_hbm.at[bat, pl.ds(seq0 + HEAD, B_PER_W - HEAD)],
                     idx_v.at[pl.ds(HEAD, B_PER_W - HEAD)], isem)

    def idx_slc(c):
        return idx_v.at[pl.ds(c * CHUNK, CHUNK)]

    def out_slc(c):
        return out_hbm.at[bat, pl.ds(seq0 + c * CHUNK, CHUNK)]

    def start_gather(c, b):
        pltpu.async_copy(table_hbm.at[idx_slc(c)], rows_v.at[b], gsem.at[b])

    def wait_gather(c, b):
        pltpu.make_async_copy(
            table_hbm.at[idx_slc(c)], rows_v.at[b], gsem.at[b]).wait()

    def start_wb(c, b):
        pltpu.async_copy(rows_v.at[b], out_slc(c), osem.at[b])

    def wait_wb(c, b):
        pltpu.make_async_copy(rows_v.at[b], out_slc(c), osem.at[b]).wait()

    for b in range(LOOKAHEAD):
        start_gather(b, b)

    pltpu.make_async_copy(
        idx_hbm.at[bat, pl.ds(seq0 + 128, B_PER_W - 128)],
        idx_v.at[pl.ds(128, B_PER_W - 128)], isem).wait()

    start_gather(LOOKAHEAD, LOOKAHEAD % NBUF)
    wait_gather(0, 0)
    start_wb(0, 0)

    @pl.loop(1, N_CHUNKS - LOOKAHEAD, step=NBUF)
    def _steady(g):
        for b in range(NBUF):
            j = g + b
            bj = (1 + b) % NBUF
            bp = b
            wait_wb(j - 1, bp)
            start_gather(j + LOOKAHEAD, bp)
            wait_gather(j, bj)
            start_wb(j, bj)

    for j in range(N_CHUNKS - LOOKAHEAD, N_CHUNKS):
        wait_wb(j - 1, (j - 1) % NBUF)
        wait_gather(j, j % NBUF)
        start_wb(j, j % NBUF)
    wait_wb(N_CHUNKS - 1, (N_CHUNKS - 1) % NBUF)


@jax.jit
def kernel(input_ids, embedding_table):
    idx = input_ids.astype(jnp.int32)
    mesh = plsc.VectorSubcoreMesh(core_axis_name="c", subcore_axis_name="s")
    return pl.kernel(
        _gather_body,
        out_type=jax.ShapeDtypeStruct((BATCH, SEQ_LEN, EMBED_DIM), jnp.float32),
        mesh=mesh,
        scratch_types=[
            pltpu.VMEM((B_PER_W,), jnp.int32),
            pltpu.VMEM((NBUF, CHUNK, EMBED_DIM), jnp.float32),
            pltpu.SemaphoreType.DMA((NBUF,)),
            pltpu.SemaphoreType.DMA((NBUF,)),
            pltpu.SemaphoreType.DMA,
        ],
    )(embedding_table, idx)
